# baseline (speedup 1.0000x reference)
import functools

import jax
import jax.numpy as jnp
from jax import lax
from jax.experimental import pallas as pl
from jax.experimental.pallas import tpu as pltpu, tpu_sc as plsc

SINK = 4
RECENT = 64
KEEP_RATIO = 0.5
ALPHA = 0.2

_NC = 2
_NS = 16
_NW = _NC * _NS

_LEVELS = (
    (521, 21, None, None, None),
    (1024, 11, 0x3FF, 21, 10),
    (2048, 0, 0x7FF, 11, 11),
)


def _tc_body(w_ref, scores_ref, budg_ref, *, H, L_q, L_kv):
    middle_len = L_kv - (SINK + RECENT)
    total_keep = int(L_kv * KEEP_RATIO)
    middle_budget = max(total_keep - (SINK + RECENT), 0)
    total_middle_budget = middle_budget * H
    min_budget = max(int(middle_len * KEEP_RATIO * ALPHA), 1)

    w = w_ref[0]
    scores_ref[0] = jnp.sum(w, axis=1)
    ent = -jnp.sum(w * jnp.log(w + 1e-8), axis=2)
    head_ent = jnp.mean(ent, axis=1, keepdims=True)

    alloc = head_ent / (jnp.sum(head_ent) + 1e-8)
    budgets = jnp.round(alloc * total_middle_budget).astype(jnp.int32)
    budgets = jnp.maximum(budgets, min_budget)
    diff = total_middle_budget - jnp.sum(budgets)
    adj = jnp.floor_divide(diff, H)
    budgets = budgets + adj
    r = diff - adj * H
    ridx = lax.broadcasted_iota(jnp.int32, (H, 1), 0)
    budgets = budgets + jnp.where(ridx < r, 1, 0)
    budgets = jnp.clip(budgets, 1, middle_len)
    budg_ref[0] = budgets.reshape(1, H)


def _tc_pass(attn_weights):
    B, H, L_q, L_kv = attn_weights.shape
    body = functools.partial(_tc_body, H=H, L_q=L_q, L_kv=L_kv)
    return pl.pallas_call(
        body,
        grid=(B,),
        in_specs=[pl.BlockSpec((1, H, L_q, L_kv), lambda b: (b, 0, 0, 0))],
        out_specs=[
            pl.BlockSpec((1, H, L_kv), lambda b: (b, 0, 0)),
            pl.BlockSpec((1, 1, H), lambda b: (b, 0, 0)),
        ],
        out_shape=[
            jax.ShapeDtypeStruct((B, H, L_kv), jnp.float32),
            jax.ShapeDtypeStruct((B, 1, H), jnp.int32),
        ],
    )(attn_weights)


def _sc_select_body(
    scores_hbm, budg_hbm, out_hbm, sbuf, hist, kref, pref, sem, *, L_kv, rpw
):
    nv_mid = (L_kv - RECENT) // 16
    nv_all = L_kv // 16
    nb_max = max(lv[0] for lv in _LEVELS)

    wid = lax.axis_index("s") * _NC + lax.axis_index("c")
    base = wid * rpw
    in_copy = pltpu.async_copy(scores_hbm.at[pl.ds(base, rpw)], sbuf, sem)
    pltpu.sync_copy(budg_hbm.at[pl.ds(base, rpw)], kref.at[pl.ds(0, rpw)])

    lane = lax.iota(jnp.int32, 16)
    ones_i = jnp.ones((16,), jnp.int32)
    zeros_i = jnp.zeros((16,), jnp.int32)

    def get_bits(r, v):
        return plsc.bitcast(sbuf[r, pl.ds(v * 16, 16)], jnp.int32)

    @plsc.parallel_loop(0, nb_max, unroll=8)
    def _(f):
        hist[pl.ds(f * 16, 16)] = zeros_i

    pref[pl.ds(0, 16)] = zeros_i
    pref[pl.ds(16, 16)] = zeros_i
    kref[pl.ds(16, 16)] = zeros_i
    in_copy.wait()

    for lvl, (nb, dshift, dmask, pshift, pwidth) in enumerate(_LEVELS):
        def srow(r, _):
            p_r = pref[pl.ds(r, 16)][0]
            idx0 = (nb - 1) * 16 + r

            def scat(v, edge):
                bits = get_bits(r, v)
                if dshift > 0:
                    digit = lax.shift_right_logical(bits, dshift)
                else:
                    digit = bits
                if dmask is not None:
                    digit = lax.bitwise_and(digit, dmask)
                idx = idx0 - lax.shift_left(digit, 4)
                part = None
                if pshift is not None:
                    part = lax.shift_right_logical(bits, pshift) == p_r
                if edge:
                    vm = lane >= SINK
                    part = vm if part is None else part & vm
                plsc.addupdate_scatter(hist, [idx], ones_i, mask=part)

            scat(0, True)

            @plsc.parallel_loop(1, nv_mid, unroll=16)
            def _(v):
                scat(v, False)

            return 0

        lax.fori_loop(0, rpw, srow, 0)

        kv = kref[pl.ds(0, 16)]

        @plsc.parallel_loop(0, nb, unroll=8, carry=(zeros_i, zeros_i, zeros_i))
        def scan_out(f, carry):
            acc, idxcnt, above = carry
            h = hist[pl.ds(f * 16, 16)]
            hist[pl.ds(f * 16, 16)] = zeros_i
            acc2 = acc + h
            lt = acc2 < kv
            idxcnt = idxcnt + jnp.where(lt, 1, 0)
            above = above + jnp.where(lt, h, 0)
            return acc2, idxcnt, above

        _, idxcnt, above = scan_out
        digit_sel = (nb - 1) - idxcnt
        kref[pl.ds(0, 16)] = kv - above
        if lvl == 0:
            pref[pl.ds(0, 16)] = digit_sel
        else:
            pref[pl.ds(0, 16)] = (
                lax.shift_left(pref[pl.ds(0, 16)], pwidth) | digit_sel
            )

    ones_f = jnp.ones((16,), jnp.float32)

    def mrow(r, _):
        t_r = pref[pl.ds(r, 16)][0]
        need_r = kref[pl.ds(r, 16)][0]

        def mask_v(v, ct, edge):
            bits = get_bits(r, v)
            gt = bits > t_r
            tie = bits == t_r
            if edge:
                vm = lane >= SINK
                gt = gt & vm
                tie = tie & vm
            tie_i = jnp.where(tie, 1, 0)
            excl = plsc.cumsum(tie_i) - tie_i
            keep_t = tie & ((ct + excl) < need_r)
            keep = gt | keep_t
            if edge:
                keep = keep | (lane < SINK)
            sbuf[r, pl.ds(v * 16, 16)] = jnp.where(keep, 1.0, 0.0).astype(
                jnp.float32
            )
            return ct + plsc.all_reduce_population_count(tie)

        ct0 = mask_v(0, zeros_i, True)

        @plsc.parallel_loop(1, nv_mid, unroll=8, carry=ct0)
        def _(v, ct):
            return mask_v(v, ct, False)

        for v in range(nv_mid, nv_all):
            sbuf[r, pl.ds(v * 16, 16)] = ones_f
        pltpu.async_copy(
            sbuf.at[pl.ds(r, 1)], out_hbm.at[pl.ds(base + r, 1)], sem
        )
        return 0

    lax.fori_loop(0, rpw, mrow, 0)

    def drain(r, _):
        pltpu.make_async_copy(
            sbuf.at[pl.ds(r, 1)], out_hbm.at[pl.ds(base + r, 1)], sem
        ).wait()
        return 0

    lax.fori_loop(0, rpw, drain, 0)


def _sc_select(scores_flat, budg_flat):
    R, L_kv = scores_flat.shape
    rpw = R // _NW
    mesh = plsc.VectorSubcoreMesh(core_axis_name="c", subcore_axis_name="s")
    body = functools.partial(_sc_select_body, L_kv=L_kv, rpw=rpw)
    return pl.kernel(
        body,
        out_type=jax.ShapeDtypeStruct((R, L_kv), jnp.float32),
        mesh=mesh,
        scratch_types=[
            pltpu.VMEM((rpw, L_kv), jnp.float32),
            pltpu.VMEM((2048 * 16,), jnp.int32),
            pltpu.VMEM((32,), jnp.int32),
            pltpu.VMEM((32,), jnp.int32),
            pltpu.SemaphoreType.DMA,
        ],
        compiler_params=pltpu.CompilerParams(needs_layout_passes=False),
    )(scores_flat, budg_flat)


def kernel(attn_weights):
    B, H, L_q, L_kv = attn_weights.shape
    scores, budgets = _tc_pass(attn_weights)
    mask_f = _sc_select(scores.reshape(B * H, L_kv), budgets.reshape(B * H))
    return mask_f.astype(jnp.bool_).reshape(B, H, L_kv)


# speedup vs baseline: 1.0197x; 1.0197x over previous
import functools

import jax
import jax.numpy as jnp
from jax import lax
from jax.experimental import pallas as pl
from jax.experimental.pallas import tpu as pltpu, tpu_sc as plsc

SINK = 4
RECENT = 64
KEEP_RATIO = 0.5
ALPHA = 0.2

_NC = 2
_NS = 16
_NW = _NC * _NS

_LEVELS = (
    (521, 21, None, None, None),
    (1024, 11, 0x3FF, 21, 10),
    (2048, 0, 0x7FF, 11, 11),
)


def _tc_body(w_ref, scores_ref, budg_ref, *, H, L_q, L_kv):
    middle_len = L_kv - (SINK + RECENT)
    total_keep = int(L_kv * KEEP_RATIO)
    middle_budget = max(total_keep - (SINK + RECENT), 0)
    total_middle_budget = middle_budget * H
    min_budget = max(int(middle_len * KEEP_RATIO * ALPHA), 1)

    w = w_ref[0]
    scores_ref[0] = jnp.sum(w, axis=1)
    ent = -jnp.sum(w * jnp.log(w + 1e-8), axis=2)
    head_ent = jnp.mean(ent, axis=1, keepdims=True)

    alloc = head_ent / (jnp.sum(head_ent) + 1e-8)
    budgets = jnp.round(alloc * total_middle_budget).astype(jnp.int32)
    budgets = jnp.maximum(budgets, min_budget)
    diff = total_middle_budget - jnp.sum(budgets)
    adj = jnp.floor_divide(diff, H)
    budgets = budgets + adj
    r = diff - adj * H
    ridx = lax.broadcasted_iota(jnp.int32, (H, 1), 0)
    budgets = budgets + jnp.where(ridx < r, 1, 0)
    budgets = jnp.clip(budgets, 1, middle_len)
    budg_ref[0] = budgets.reshape(1, H)


def _tc_pass(attn_weights):
    B, H, L_q, L_kv = attn_weights.shape
    body = functools.partial(_tc_body, H=H, L_q=L_q, L_kv=L_kv)
    return pl.pallas_call(
        body,
        grid=(B,),
        in_specs=[pl.BlockSpec((1, H, L_q, L_kv), lambda b: (b, 0, 0, 0))],
        out_specs=[
            pl.BlockSpec((1, H, L_kv), lambda b: (b, 0, 0)),
            pl.BlockSpec((1, 1, H), lambda b: (b, 0, 0)),
        ],
        out_shape=[
            jax.ShapeDtypeStruct((B, H, L_kv), jnp.float32),
            jax.ShapeDtypeStruct((B, 1, H), jnp.int32),
        ],
    )(attn_weights)


def _sc_select_body(
    scores_hbm, budg_hbm, out_hbm, sbuf, hist, kref, pref, sem, *, L_kv, rpw
):
    nv_mid = (L_kv - RECENT) // 16
    nv_all = L_kv // 16
    nb_max = max(lv[0] for lv in _LEVELS)

    wid = lax.axis_index("s") * _NC + lax.axis_index("c")
    base = wid * rpw
    pltpu.sync_copy(scores_hbm.at[pl.ds(base, rpw)], sbuf)
    pltpu.sync_copy(budg_hbm.at[pl.ds(base, rpw)], kref.at[pl.ds(0, rpw)])

    lane = lax.iota(jnp.int32, 16)
    ones_i = jnp.ones((16,), jnp.int32)
    zeros_i = jnp.zeros((16,), jnp.int32)

    def get_bits(r, v):
        return plsc.bitcast(sbuf[r, pl.ds(v * 16, 16)], jnp.int32)

    @plsc.parallel_loop(0, nb_max, unroll=8)
    def _(f):
        hist[pl.ds(f * 16, 16)] = zeros_i

    pref[pl.ds(0, 16)] = zeros_i
    pref[pl.ds(16, 16)] = zeros_i
    kref[pl.ds(16, 16)] = zeros_i

    for lvl, (nb, dshift, dmask, pshift, pwidth) in enumerate(_LEVELS):
        def srow(r, _):
            p_r = pref[pl.ds(r, 16)][0]
            idx0 = (nb - 1) * 16 + r

            def scat(v, edge):
                bits = get_bits(r, v)
                if dshift > 0:
                    digit = lax.shift_right_logical(bits, dshift)
                else:
                    digit = bits
                if dmask is not None:
                    digit = lax.bitwise_and(digit, dmask)
                idx = idx0 - lax.shift_left(digit, 4)
                part = None
                if pshift is not None:
                    part = lax.shift_right_logical(bits, pshift) == p_r
                if edge:
                    vm = lane >= SINK
                    part = vm if part is None else part & vm
                plsc.addupdate_scatter(hist, [idx], ones_i, mask=part)

            scat(0, True)

            @plsc.parallel_loop(1, nv_mid, unroll=8)
            def _(v):
                scat(v, False)

            return 0

        lax.fori_loop(0, rpw, srow, 0)

        kv = kref[pl.ds(0, 16)]

        @plsc.parallel_loop(0, nb, unroll=8, carry=(zeros_i, zeros_i, zeros_i))
        def scan_out(f, carry):
            acc, idxcnt, above = carry
            h = hist[pl.ds(f * 16, 16)]
            hist[pl.ds(f * 16, 16)] = zeros_i
            acc2 = acc + h
            lt = acc2 < kv
            idxcnt = idxcnt + jnp.where(lt, 1, 0)
            above = above + jnp.where(lt, h, 0)
            return acc2, idxcnt, above

        _, idxcnt, above = scan_out
        digit_sel = (nb - 1) - idxcnt
        kref[pl.ds(0, 16)] = kv - above
        if lvl == 0:
            pref[pl.ds(0, 16)] = digit_sel
        else:
            pref[pl.ds(0, 16)] = (
                lax.shift_left(pref[pl.ds(0, 16)], pwidth) | digit_sel
            )

    ones_f = jnp.ones((16,), jnp.float32)

    def mrow(r, _):
        t_r = pref[pl.ds(r, 16)][0]
        need_r = kref[pl.ds(r, 16)][0]

        def mask_v(v, ct, edge):
            bits = get_bits(r, v)
            gt = bits > t_r
            tie = bits == t_r
            if edge:
                vm = lane >= SINK
                gt = gt & vm
                tie = tie & vm
            tie_i = jnp.where(tie, 1, 0)
            excl = plsc.cumsum(tie_i) - tie_i
            keep_t = tie & ((ct + excl) < need_r)
            keep = gt | keep_t
            if edge:
                keep = keep | (lane < SINK)
            sbuf[r, pl.ds(v * 16, 16)] = jnp.where(keep, 1.0, 0.0).astype(
                jnp.float32
            )
            return ct + plsc.all_reduce_population_count(tie)

        ct0 = mask_v(0, zeros_i, True)

        @plsc.parallel_loop(1, nv_mid, unroll=4, carry=ct0)
        def _(v, ct):
            return mask_v(v, ct, False)

        for v in range(nv_mid, nv_all):
            sbuf[r, pl.ds(v * 16, 16)] = ones_f
        pltpu.async_copy(
            sbuf.at[pl.ds(r, 1)], out_hbm.at[pl.ds(base + r, 1)], sem
        )
        return 0

    lax.fori_loop(0, rpw, mrow, 0)

    def drain(r, _):
        pltpu.make_async_copy(
            sbuf.at[pl.ds(r, 1)], out_hbm.at[pl.ds(base + r, 1)], sem
        ).wait()
        return 0

    lax.fori_loop(0, rpw, drain, 0)


def _sc_select(scores_flat, budg_flat):
    R, L_kv = scores_flat.shape
    rpw = R // _NW
    mesh = plsc.VectorSubcoreMesh(core_axis_name="c", subcore_axis_name="s")
    body = functools.partial(_sc_select_body, L_kv=L_kv, rpw=rpw)
    return pl.kernel(
        body,
        out_type=jax.ShapeDtypeStruct((R, L_kv), jnp.float32),
        mesh=mesh,
        scratch_types=[
            pltpu.VMEM((rpw, L_kv), jnp.float32),
            pltpu.VMEM((2048 * 16,), jnp.int32),
            pltpu.VMEM((32,), jnp.int32),
            pltpu.VMEM((32,), jnp.int32),
            pltpu.SemaphoreType.DMA,
        ],
        compiler_params=pltpu.CompilerParams(needs_layout_passes=False),
    )(scores_flat, budg_flat)


def kernel(attn_weights):
    B, H, L_q, L_kv = attn_weights.shape
    scores, budgets = _tc_pass(attn_weights)
    mask_f = _sc_select(scores.reshape(B * H, L_kv), budgets.reshape(B * H))
    return mask_f.astype(jnp.bool_).reshape(B, H, L_kv)
